# Initial kernel scaffold; baseline (speedup 1.0000x reference)
import jax
import jax.numpy as jnp
from jax.experimental import pallas as pl


def kernel(inputs, shared_weights):
    raise NotImplementedError("write your pallas kernel here")


# speedup vs baseline: 1.3117x; 1.3117x over previous
import functools

import jax
import jax.numpy as jnp
from jax import lax
from jax.experimental import pallas as pl
from jax.experimental.pallas import tpu as pltpu, tpu_sc as plsc

D = 1024
SCALE = float(D) ** 0.5
NC = 2
NS = 16
NW = NC * NS
LANES = 16


def _make_emb_kernel(n_rows: int):
    per_w = n_rows // NW
    chunk = 32
    nchunk = per_w // chunk

    mesh = plsc.VectorSubcoreMesh(
        core_axis_name="c", subcore_axis_name="s",
        num_cores=NC, num_subcores=NS,
    )

    @functools.partial(
        pl.kernel,
        out_type=jax.ShapeDtypeStruct((n_rows, D), jnp.float32),
        mesh=mesh,
        compiler_params=pltpu.CompilerParams(needs_layout_passes=False),
        scratch_types=[
            pltpu.VMEM((per_w,), jnp.int32),
            pltpu.VMEM((chunk, D), jnp.float32),
            pltpu.VMEM((chunk, D), jnp.float32),
            pltpu.SemaphoreType.DMA,
            pltpu.SemaphoreType.DMA,
            pltpu.SemaphoreType.DMA,
            pltpu.SemaphoreType.DMA,
        ],
    )
    def emb(idx_hbm, table_hbm, out_hbm, idx_v, buf0, buf1,
            gsem0, gsem1, osem0, osem1):
        wid = lax.axis_index("s") * NC + lax.axis_index("c")
        base = wid * per_w
        pltpu.sync_copy(idx_hbm.at[pl.ds(base, per_w)], idx_v)

        bufs = (buf0, buf1)
        gsems = (gsem0, gsem1)
        osems = (osem0, osem1)

        def start_gather(c):
            p = c & 1
            return pltpu.async_copy(
                table_hbm.at[idx_v.at[pl.ds(c * chunk, chunk)]],
                bufs[p], gsems[p])

        def start_out(c):
            p = c & 1
            return pltpu.async_copy(
                bufs[p], out_hbm.at[pl.ds(base + c * chunk, chunk)],
                osems[p])

        def compute(c):
            buf = bufs[c & 1]

            def row_body(r, carry):
                iv = plsc.load_gather(
                    idx_v, [jnp.full((LANES,), c * chunk, jnp.int32) + r])
                sc = jnp.where(iv == 0, 0.0, SCALE).astype(jnp.float32)
                for j in range(D // LANES):
                    buf[r, pl.ds(j * LANES, LANES)] = (
                        buf[r, pl.ds(j * LANES, LANES)] * sc)
                return carry

            lax.fori_loop(0, chunk, row_body, 0)

        ghandle = start_gather(0)
        ohandles = [None, None]
        for c in range(nchunk):
            p = c & 1
            ghandle.wait()
            if c + 1 < nchunk:
                q = (c + 1) & 1
                if ohandles[q] is not None:
                    ohandles[q].wait()
                ghandle = start_gather(c + 1)
            compute(c)
            ohandles[p] = start_out(c)
        for h in ohandles:
            if h is not None:
                h.wait()

    return emb


@jax.jit
def kernel(inputs, shared_weights):
    b, s = inputs.shape
    n = b * s
    flat_idx = inputs.reshape(n).astype(jnp.int32)
    emb = _make_emb_kernel(n)
    out = emb(flat_idx, shared_weights)
    return out.reshape(b, s, shared_weights.shape[1])


# speedup vs baseline: 1.3536x; 1.0320x over previous
import functools

import jax
import jax.numpy as jnp
from jax import lax
from jax.experimental import pallas as pl
from jax.experimental.pallas import tpu as pltpu, tpu_sc as plsc

D = 1024
SCALE = float(D) ** 0.5
NC = 2
NS = 16
NW = NC * NS
LANES = 16


def _make_emb_kernel(n_rows: int):
    per_w = n_rows // NW
    chunk = 32
    nchunk = per_w // chunk

    mesh = plsc.VectorSubcoreMesh(
        core_axis_name="c", subcore_axis_name="s",
        num_cores=NC, num_subcores=NS,
    )

    @functools.partial(
        pl.kernel,
        out_type=jax.ShapeDtypeStruct((n_rows, D), jnp.float32),
        mesh=mesh,
        compiler_params=pltpu.CompilerParams(needs_layout_passes=False),
        scratch_types=[
            pltpu.VMEM((per_w,), jnp.int32),
            pltpu.VMEM((chunk, D), jnp.float32),
            pltpu.VMEM((chunk, D), jnp.float32),
            pltpu.SemaphoreType.DMA,
            pltpu.SemaphoreType.DMA,
            pltpu.SemaphoreType.DMA,
            pltpu.SemaphoreType.DMA,
        ],
    )
    def emb(idx_hbm, table_hbm, out_hbm, idx_v, buf0, buf1,
            gsem0, gsem1, osem0, osem1):
        wid = lax.axis_index("s") * NC + lax.axis_index("c")
        base = wid * per_w
        pltpu.sync_copy(idx_hbm.at[pl.ds(base, per_w)], idx_v)

        bufs = (buf0, buf1)
        gsems = (gsem0, gsem1)
        osems = (osem0, osem1)

        def start_gather(c):
            p = c & 1
            return pltpu.async_copy(
                table_hbm.at[idx_v.at[pl.ds(c * chunk, chunk)]],
                bufs[p], gsems[p])

        def start_out(c):
            p = c & 1
            return pltpu.async_copy(
                bufs[p], out_hbm.at[pl.ds(base + c * chunk, chunk)],
                osems[p])

        def compute(c):
            buf = bufs[c & 1]

            def grp_body(g, carry):
                base_r = g * LANES
                iv = idx_v[pl.ds(c * chunk + base_r, LANES)]
                sv = jnp.where(iv == 0, 0.0, SCALE).astype(jnp.float32)

                def row_body(rr, carry2):
                    bc = jnp.take_along_axis(
                        sv, jnp.full((LANES,), rr, jnp.int32), axis=0)
                    r = base_r + rr
                    for j in range(D // LANES):
                        buf[r, pl.ds(j * LANES, LANES)] = (
                            buf[r, pl.ds(j * LANES, LANES)] * bc)
                    return carry2

                lax.fori_loop(0, LANES, row_body, 0)
                return carry

            lax.fori_loop(0, chunk // LANES, grp_body, 0)

        ghandle = start_gather(0)
        ohandles = [None, None]
        for c in range(nchunk):
            p = c & 1
            ghandle.wait()
            if c + 1 < nchunk:
                q = (c + 1) & 1
                if ohandles[q] is not None:
                    ohandles[q].wait()
                ghandle = start_gather(c + 1)
            compute(c)
            ohandles[p] = start_out(c)
        for h in ohandles:
            if h is not None:
                h.wait()

    return emb


@jax.jit
def kernel(inputs, shared_weights):
    b, s = inputs.shape
    n = b * s
    flat_idx = inputs.reshape(n).astype(jnp.int32)
    emb = _make_emb_kernel(n)
    out = emb(flat_idx, shared_weights)
    return out.reshape(b, s, shared_weights.shape[1])
